# Initial kernel scaffold; baseline (speedup 1.0000x reference)
import jax
import jax.numpy as jnp
from jax.experimental import pallas as pl


def kernel(x, edge_index, edge_attr, batch, atom_tab, vn_emb, eps, W1, b1, bn1_g, bn1_b, W2, b2, bond_tab, bn_g, bn_b, vW1, vb1, vbn1_g, vbn1_b, vW2, vb2, vbn2_g, vbn2_b):
    raise NotImplementedError("write your pallas kernel here")


# speedup vs baseline: 5.4926x; 5.4926x over previous
import functools

import jax
import jax.numpy as jnp
from jax import lax
from jax.experimental import pallas as pl
from jax.experimental.pallas import tpu as pltpu
from jax.experimental.pallas import tpu_sc as plsc

N = 10000
E = 320000
D = 128
L = 3
G = 128
NP = 10240
CH = 80
NC = 2
NS = 16
NW = NC * NS
EPW = E // NW
NCHUNK = EPW // CH
RPT = NP // NS
B = 2000
NB = N // B

_F32 = jnp.float32
_HI = jax.lax.Precision.HIGHEST


def _p0_body(x_ref, atom_ref, vne_ref, hin_ref):
    h = jnp.broadcast_to(vne_ref[...], (B, D))
    for f in range(9):
        oh = (x_ref[:, f:f + 1]
              == lax.broadcasted_iota(jnp.int32, (B, 64), 1)).astype(_F32)
        h = h + jnp.dot(oh, atom_ref[f * 64:(f + 1) * 64, :],
                        preferred_element_type=_F32, precision=_HI)
    hin_ref[...] = h


def _etab_body(bond_ref, m_ref, etab_ref):
    etab_ref[...] = jnp.dot(m_ref[...], bond_ref[...],
                            preferred_element_type=_F32, precision=_HI)


def _eidx_body(attr_ref, eidx_ref):
    a = attr_ref[...]
    eidx_ref[...] = a[0] * 64 + a[1] * 8 + a[2]


def _p1_body(hin_ref, a0_ref, a1_ref, brow_ref, eps_ref, w1_ref, b1_ref,
             z1_ref, s1_ref, q1_ref, vns_ref):
    i = pl.program_id(0)
    hin = hin_ref[...]
    z = (1.0 + eps_ref[...]) * hin + a0_ref[...] + a1_ref[...]
    z1 = jnp.dot(z, w1_ref[...], preferred_element_type=_F32) + b1_ref[...]
    z1_ref[...] = z1
    oht = (brow_ref[0]
           == lax.broadcasted_iota(jnp.int32, (G, B), 0)).astype(_F32)
    part = jnp.dot(oht, hin, preferred_element_type=_F32, precision=_HI)

    @pl.when(i == 0)
    def _():
        s1_ref[...] = jnp.zeros_like(s1_ref)
        q1_ref[...] = jnp.zeros_like(q1_ref)
        vns_ref[...] = jnp.zeros_like(vns_ref)

    s1_ref[...] += jnp.sum(z1, axis=0, keepdims=True)
    q1_ref[...] += jnp.sum(z1 * z1, axis=0, keepdims=True)
    vns_ref[...] += part


def _p2_body(vns_ref, vn_ref, ng_ref, vw1_ref, vb1_ref, vg1_ref, vbe1_ref,
             vw2_ref, vb2_ref, vg2_ref, vbe2_ref, bond_ref, m_ref,
             vno_ref, etab_ref):
    mask = (lax.broadcasted_iota(jnp.int32, (G, 1), 0)
            < ng_ref[...]).astype(_F32)
    cnt = ng_ref[...].astype(_F32)

    def bnm(z, g, b):
        m = jnp.sum(z * mask, axis=0, keepdims=True) / cnt
        zc = z - m
        v = jnp.sum(zc * zc * mask, axis=0, keepdims=True) / cnt
        return zc * jax.lax.rsqrt(v + 1e-5) * g + b

    tmp = vns_ref[...] + vn_ref[...]
    t = jnp.dot(tmp, vw1_ref[...], preferred_element_type=_F32) + vb1_ref[...]
    t = jnp.maximum(bnm(t, vg1_ref[...], vbe1_ref[...]), 0.0)
    t = jnp.dot(t, vw2_ref[...], preferred_element_type=_F32) + vb2_ref[...]
    vno_ref[...] = jnp.maximum(bnm(t, vg2_ref[...], vbe2_ref[...]), 0.0)
    etab_ref[...] = jnp.dot(m_ref[...], bond_ref[...],
                            preferred_element_type=_F32, precision=_HI)


def _p3_body(z1_ref, s1_ref, q1_ref, g1_ref, be1_ref, w2_ref, b2_ref,
             z2_ref, s2_ref, q2_ref):
    i = pl.program_id(0)
    m = s1_ref[...] * (1.0 / N)
    v = q1_ref[...] * (1.0 / N) - m * m
    z1n = jnp.maximum((z1_ref[...] - m) * jax.lax.rsqrt(v + 1e-5)
                      * g1_ref[...] + be1_ref[...], 0.0)
    z2 = jnp.dot(z1n, w2_ref[...], preferred_element_type=_F32) + b2_ref[...]
    z2_ref[...] = z2

    @pl.when(i == 0)
    def _():
        s2_ref[...] = jnp.zeros_like(s2_ref)
        q2_ref[...] = jnp.zeros_like(q2_ref)

    s2_ref[...] += jnp.sum(z2, axis=0, keepdims=True)
    q2_ref[...] += jnp.sum(z2 * z2, axis=0, keepdims=True)


def _p4_body(z2_ref, s2_ref, q2_ref, go_ref, bo_ref, b2d_ref, vno_ref,
             hout_ref):
    m = s2_ref[...] * (1.0 / N)
    v = q2_ref[...] * (1.0 / N) - m * m
    h = jnp.maximum((z2_ref[...] - m) * jax.lax.rsqrt(v + 1e-5)
                    * go_ref[...] + bo_ref[...], 0.0)
    oh = (b2d_ref[...]
          == lax.broadcasted_iota(jnp.int32, (B, G), 1)).astype(_F32)
    hout_ref[...] = h + jnp.dot(oh, vno_ref[...],
                                preferred_element_type=_F32, precision=_HI)


def _p4f_body(z2_ref, s2_ref, q2_ref, go_ref, bo_ref, out_ref):
    m = s2_ref[...] * (1.0 / N)
    v = q2_ref[...] * (1.0 / N) - m * m
    out_ref[...] = ((z2_ref[...] - m) * jax.lax.rsqrt(v + 1e-5)
                    * go_ref[...] + bo_ref[...])


def _edge_body(hin, etab, src, dst, eidx, out,
               acc, zrow, idxs, idxd, idxe, ra, rb, sem1, sem2):
    c = lax.axis_index("c")
    s = lax.axis_index("s")
    wid = s * NC + c

    def _zr(r, carry):
        for k in range(D // 16):
            zrow[r, pl.ds(k * 16, 16)] = jnp.zeros((16,), _F32)
        return carry
    lax.fori_loop(0, CH, _zr, 0)
    for k in range(RPT // CH):
        pltpu.sync_copy(zrow, acc.at[pl.ds(s * RPT + k * CH, CH)])
    plsc.subcore_barrier()

    def _chunk(i, carry):
        base = wid * EPW + i * CH
        pltpu.sync_copy(src.at[pl.ds(base, CH)], idxs)
        pltpu.sync_copy(dst.at[pl.ds(base, CH)], idxd)
        pltpu.sync_copy(eidx.at[pl.ds(base, CH)], idxe)
        cp1 = pltpu.async_copy(hin.at[idxs], ra, sem1)
        cp2 = pltpu.async_copy(etab.at[idxe], rb, sem2)
        cp1.wait()
        cp2.wait()

        def _row(r, cc):
            for k in range(D // 16):
                sl = pl.ds(k * 16, 16)
                ra[r, sl] = jnp.maximum(ra[r, sl] + rb[r, sl], 0.0)
            return cc
        lax.fori_loop(0, CH, _row, 0)
        pltpu.sync_copy(ra, acc.at[idxd], add=True)
        return carry
    lax.fori_loop(0, NCHUNK, _chunk, 0)
    plsc.subcore_barrier()
    for k in range(RPT // CH):
        r0 = s * RPT + k * CH
        pltpu.sync_copy(acc.at[pl.ds(r0, CH)], out.at[c, pl.ds(r0, CH)])


@functools.lru_cache(maxsize=1)
def _make_edge_call():
    return pl.kernel(
        _edge_body,
        out_type=jax.ShapeDtypeStruct((NC, NP, D), _F32),
        mesh=plsc.VectorSubcoreMesh(core_axis_name="c",
                                    subcore_axis_name="s"),
        scratch_types=[
            pltpu.VMEM_SHARED((NP, D), _F32),
            pltpu.VMEM((CH, D), _F32),
            pltpu.VMEM((CH,), jnp.int32),
            pltpu.VMEM((CH,), jnp.int32),
            pltpu.VMEM((CH,), jnp.int32),
            pltpu.VMEM((CH, D), _F32),
            pltpu.VMEM((CH, D), _F32),
            pltpu.SemaphoreType.DMA,
            pltpu.SemaphoreType.DMA,
        ],
    )


def _full(shape):
    nd = len(shape)
    return pl.BlockSpec(shape, lambda i, _nd=nd: (0,) * _nd)


def _rows(shape):
    nd = len(shape)
    return pl.BlockSpec(shape, lambda i, _nd=nd: (i,) + (0,) * (_nd - 1))


def kernel(x, edge_index, edge_attr, batch, atom_tab, vn_emb, eps, W1, b1,
           bn1_g, bn1_b, W2, b2, bond_tab, bn_g, bn_b, vW1, vb1, vbn1_g,
           vbn1_b, vW2, vb2, vbn2_g, vbn2_b):
    x = x.astype(jnp.int32)
    src = edge_index[0].astype(jnp.int32)
    dst = edge_index[1].astype(jnp.int32)
    batch = batch.astype(jnp.int32)
    atom2 = atom_tab.reshape(9 * 64, D)
    attr3 = edge_attr.astype(jnp.int32).T.reshape(3, E // 128, 128)
    b2d = batch[:, None]
    brow = batch.reshape(NB, 1, B)
    ng = (batch[-1] + 1).reshape(1, 1)
    ii = jnp.arange(512)
    m_sel = jnp.concatenate([
        (ii[:, None] // 64 == jnp.arange(8)[None, :]).astype(_F32),
        ((ii[:, None] // 8) % 8 == jnp.arange(8)[None, :]).astype(_F32),
        (ii[:, None] % 8 == jnp.arange(8)[None, :]).astype(_F32)], axis=1)

    hin = pl.pallas_call(
        _p0_body,
        grid=(NB,),
        in_specs=[_rows((B, 9)), _full((9 * 64, D)), _full((1, D))],
        out_specs=_rows((B, D)),
        out_shape=jax.ShapeDtypeStruct((N, D), _F32),
    )(x, atom2, vn_emb)

    etab_call = pl.pallas_call(
        _etab_body,
        out_shape=jax.ShapeDtypeStruct((512, D), _F32))
    etab = etab_call(bond_tab[0].reshape(24, D), m_sel)
    eidx = pl.pallas_call(
        _eidx_body,
        out_shape=jax.ShapeDtypeStruct((E // 128, 128), jnp.int32),
    )(attr3).reshape(E)

    vn = jnp.broadcast_to(vn_emb[0], (G, D))

    p1_call = pl.pallas_call(
        _p1_body,
        grid=(NB,),
        in_specs=[_rows((B, D)), _rows((B, D)), _rows((B, D)),
                  pl.BlockSpec((1, 1, B), lambda i: (i, 0, 0)),
                  _full((1, 1)),
                  _full((D, 2 * D)), _full((1, 2 * D))],
        out_specs=[_rows((B, 2 * D)), _full((1, 2 * D)), _full((1, 2 * D)),
                   _full((G, D))],
        out_shape=[jax.ShapeDtypeStruct((N, 2 * D), _F32),
                   jax.ShapeDtypeStruct((1, 2 * D), _F32),
                   jax.ShapeDtypeStruct((1, 2 * D), _F32),
                   jax.ShapeDtypeStruct((G, D), _F32)],
    )

    p2_call = pl.pallas_call(
        _p2_body,
        out_shape=[jax.ShapeDtypeStruct((G, D), _F32),
                   jax.ShapeDtypeStruct((512, D), _F32)])

    p3_call = pl.pallas_call(
        _p3_body,
        grid=(NB,),
        in_specs=[_rows((B, 2 * D)), _full((1, 2 * D)), _full((1, 2 * D)),
                  _full((1, 2 * D)), _full((1, 2 * D)),
                  _full((2 * D, D)), _full((1, D))],
        out_specs=[_rows((B, D)), _full((1, D)), _full((1, D))],
        out_shape=[jax.ShapeDtypeStruct((N, D), _F32),
                   jax.ShapeDtypeStruct((1, D), _F32),
                   jax.ShapeDtypeStruct((1, D), _F32)],
    )

    p4_call = pl.pallas_call(
        _p4_body,
        grid=(NB,),
        in_specs=[_rows((B, D)), _full((1, D)), _full((1, D)),
                  _full((1, D)), _full((1, D)), _rows((B, 1)),
                  _full((G, D))],
        out_specs=_rows((B, D)),
        out_shape=jax.ShapeDtypeStruct((N, D), _F32),
    )

    p4f_call = pl.pallas_call(
        _p4f_body,
        grid=(NB,),
        in_specs=[_rows((B, D)), _full((1, D)), _full((1, D)),
                  _full((1, D)), _full((1, D))],
        out_specs=_rows((B, D)),
        out_shape=jax.ShapeDtypeStruct((N, D), _F32),
    )

    out = None
    for l in range(L):
        hin_p = jnp.pad(hin, ((0, NP - N), (0, 0)))
        aggp = _make_edge_call()(hin_p, etab, src, dst, eidx)
        z1, s1, q1, vns = p1_call(
            hin, aggp[0, :N], aggp[1, :N], brow, eps[l].reshape(1, 1),
            W1[l], b1[l][None, :])
        z2, s2, q2 = p3_call(z1, s1, q1, bn1_g[l][None, :],
                             bn1_b[l][None, :], W2[l], b2[l][None, :])
        if l < L - 1:
            vn, etab = p2_call(
                vns, vn, ng, vW1[l], vb1[l][None, :], vbn1_g[l][None, :],
                vbn1_b[l][None, :], vW2[l], vb2[l][None, :],
                vbn2_g[l][None, :], vbn2_b[l][None, :],
                bond_tab[l + 1].reshape(24, D), m_sel)
            hin = p4_call(z2, s2, q2, bn_g[l][None, :], bn_b[l][None, :],
                          b2d, vn)
        else:
            out = p4f_call(z2, s2, q2, bn_g[l][None, :], bn_b[l][None, :])
    return out


# speedup vs baseline: 7.8410x; 1.4276x over previous
import functools

import jax
import jax.numpy as jnp
from jax import lax
from jax.experimental import pallas as pl
from jax.experimental.pallas import tpu as pltpu
from jax.experimental.pallas import tpu_sc as plsc

N = 10000
E = 320000
D = 128
L = 3
G = 128
NP = 10240
CH = 80
NC = 2
NS = 16
NW = NC * NS
EPW = E // NW
NCHUNK = EPW // CH
RPT = NP // NS
B = 2000
NB = N // B

_F32 = jnp.float32
_HI = jax.lax.Precision.HIGHEST


def _p0_body(x_ref, atom_ref, vne_ref, hin_ref):
    h = jnp.broadcast_to(vne_ref[...], (B, D))
    for f in range(9):
        oh = (x_ref[:, f:f + 1]
              == lax.broadcasted_iota(jnp.int32, (B, 64), 1)).astype(_F32)
        h = h + jnp.dot(oh, atom_ref[f * 64:(f + 1) * 64, :],
                        preferred_element_type=_F32, precision=_HI)
    hin_ref[...] = h


def _etab_body(bond_ref, m_ref, etab_ref):
    etab_ref[...] = jnp.dot(m_ref[...], bond_ref[...],
                            preferred_element_type=_F32, precision=_HI)


def _eidx_body(attr_ref, eidx_ref):
    a = attr_ref[...]
    eidx_ref[...] = a[0] * 64 + a[1] * 8 + a[2]


def _p1_body(hin_ref, a0_ref, a1_ref, brow_ref, eps_ref, w1_ref, b1_ref,
             z1_ref, s1_ref, q1_ref, vns_ref):
    i = pl.program_id(0)
    hin = hin_ref[...]
    z = (1.0 + eps_ref[...]) * hin + a0_ref[...] + a1_ref[...]
    z1 = jnp.dot(z, w1_ref[...], preferred_element_type=_F32) + b1_ref[...]
    z1_ref[...] = z1
    oht = (brow_ref[0]
           == lax.broadcasted_iota(jnp.int32, (G, B), 0)).astype(_F32)
    part = jnp.dot(oht, hin, preferred_element_type=_F32, precision=_HI)

    @pl.when(i == 0)
    def _():
        s1_ref[...] = jnp.zeros_like(s1_ref)
        q1_ref[...] = jnp.zeros_like(q1_ref)
        vns_ref[...] = jnp.zeros_like(vns_ref)

    s1_ref[...] += jnp.sum(z1, axis=0, keepdims=True)
    q1_ref[...] += jnp.sum(z1 * z1, axis=0, keepdims=True)
    vns_ref[...] += part


def _p2_body(vns_ref, vn_ref, ng_ref, vw1_ref, vb1_ref, vg1_ref, vbe1_ref,
             vw2_ref, vb2_ref, vg2_ref, vbe2_ref, bond_ref, m_ref,
             vno_ref, etab_ref):
    mask = (lax.broadcasted_iota(jnp.int32, (G, 1), 0)
            < ng_ref[...]).astype(_F32)
    cnt = ng_ref[...].astype(_F32)

    def bnm(z, g, b):
        m = jnp.sum(z * mask, axis=0, keepdims=True) / cnt
        zc = z - m
        v = jnp.sum(zc * zc * mask, axis=0, keepdims=True) / cnt
        return zc * jax.lax.rsqrt(v + 1e-5) * g + b

    tmp = vns_ref[...] + vn_ref[...]
    t = jnp.dot(tmp, vw1_ref[...], preferred_element_type=_F32) + vb1_ref[...]
    t = jnp.maximum(bnm(t, vg1_ref[...], vbe1_ref[...]), 0.0)
    t = jnp.dot(t, vw2_ref[...], preferred_element_type=_F32) + vb2_ref[...]
    vno_ref[...] = jnp.maximum(bnm(t, vg2_ref[...], vbe2_ref[...]), 0.0)
    etab_ref[...] = jnp.dot(m_ref[...], bond_ref[...],
                            preferred_element_type=_F32, precision=_HI)


def _p3_body(z1_ref, s1_ref, q1_ref, g1_ref, be1_ref, w2_ref, b2_ref,
             z2_ref, s2_ref, q2_ref):
    i = pl.program_id(0)
    m = s1_ref[...] * (1.0 / N)
    v = q1_ref[...] * (1.0 / N) - m * m
    z1n = jnp.maximum((z1_ref[...] - m) * jax.lax.rsqrt(v + 1e-5)
                      * g1_ref[...] + be1_ref[...], 0.0)
    z2 = jnp.dot(z1n, w2_ref[...], preferred_element_type=_F32) + b2_ref[...]
    z2_ref[...] = z2

    @pl.when(i == 0)
    def _():
        s2_ref[...] = jnp.zeros_like(s2_ref)
        q2_ref[...] = jnp.zeros_like(q2_ref)

    s2_ref[...] += jnp.sum(z2, axis=0, keepdims=True)
    q2_ref[...] += jnp.sum(z2 * z2, axis=0, keepdims=True)


def _p4_body(z2_ref, s2_ref, q2_ref, go_ref, bo_ref, b2d_ref, vno_ref,
             hout_ref):
    m = s2_ref[...] * (1.0 / N)
    v = q2_ref[...] * (1.0 / N) - m * m
    h = jnp.maximum((z2_ref[...] - m) * jax.lax.rsqrt(v + 1e-5)
                    * go_ref[...] + bo_ref[...], 0.0)
    oh = (b2d_ref[...]
          == lax.broadcasted_iota(jnp.int32, (B, G), 1)).astype(_F32)
    hout_ref[...] = h + jnp.dot(oh, vno_ref[...],
                                preferred_element_type=_F32, precision=_HI)


def _p4f_body(z2_ref, s2_ref, q2_ref, go_ref, bo_ref, out_ref):
    m = s2_ref[...] * (1.0 / N)
    v = q2_ref[...] * (1.0 / N) - m * m
    out_ref[...] = ((z2_ref[...] - m) * jax.lax.rsqrt(v + 1e-5)
                    * go_ref[...] + bo_ref[...])


def _edge_body(hin, etab, src, dst, eidx, out,
               acc, sA, dA, eA, sB, dB, eB, ra0, rb0, ra1, rb1, semA, semB):
    c = lax.axis_index("c")
    s = lax.axis_index("s")
    wid = s * NC + c

    def _zr(r, carry):
        for k in range(D // 16):
            ra0[r, pl.ds(k * 16, 16)] = jnp.zeros((16,), _F32)
        return carry
    lax.fori_loop(0, CH, _zr, 0)
    for k in range(RPT // CH):
        pltpu.sync_copy(ra0, acc.at[pl.ds(s * RPT + k * CH, CH)])

    def _fetch_start(i, ixs, ixd, ixe, ra, rb, sem):
        base = wid * EPW + i * CH
        pltpu.sync_copy(src.at[pl.ds(base, CH)], ixs)
        pltpu.sync_copy(dst.at[pl.ds(base, CH)], ixd)
        pltpu.sync_copy(eidx.at[pl.ds(base, CH)], ixe)
        pltpu.async_copy(hin.at[ixs], ra, sem)
        pltpu.async_copy(etab.at[ixe], rb, sem)

    def _process(ixs, ixd, ixe, ra, rb, sem):
        pltpu.make_async_copy(hin.at[ixs], ra, sem).wait()
        pltpu.make_async_copy(etab.at[ixe], rb, sem).wait()

        def _row(r, cc):
            for k in range(D // 16):
                sl = pl.ds(k * 16, 16)
                ra[r, sl] = jnp.maximum(ra[r, sl] + rb[r, sl], 0.0)
            return cc
        lax.fori_loop(0, CH, _row, 0)
        pltpu.sync_copy(ra, acc.at[ixd], add=True)

    _fetch_start(0, sA, dA, eA, ra0, rb0, semA)
    plsc.subcore_barrier()

    def _pair(j, carry):
        i1 = 2 * j + 1
        i2 = 2 * j + 2

        @pl.when(i1 < NCHUNK)
        def _():
            _fetch_start(i1, sB, dB, eB, ra1, rb1, semB)
        _process(sA, dA, eA, ra0, rb0, semA)

        @pl.when(i2 < NCHUNK)
        def _():
            _fetch_start(i2, sA, dA, eA, ra0, rb0, semA)

        @pl.when(i1 < NCHUNK)
        def _():
            _process(sB, dB, eB, ra1, rb1, semB)
        return carry
    lax.fori_loop(0, (NCHUNK + 1) // 2, _pair, 0)
    plsc.subcore_barrier()
    for k in range(RPT // CH):
        r0 = s * RPT + k * CH
        pltpu.sync_copy(acc.at[pl.ds(r0, CH)], out.at[c, pl.ds(r0, CH)])


@functools.lru_cache(maxsize=1)
def _make_edge_call():
    return pl.kernel(
        _edge_body,
        out_type=jax.ShapeDtypeStruct((NC, NP, D), _F32),
        mesh=plsc.VectorSubcoreMesh(core_axis_name="c",
                                    subcore_axis_name="s"),
        scratch_types=[
            pltpu.VMEM_SHARED((NP, D), _F32),
            pltpu.VMEM((CH,), jnp.int32),
            pltpu.VMEM((CH,), jnp.int32),
            pltpu.VMEM((CH,), jnp.int32),
            pltpu.VMEM((CH,), jnp.int32),
            pltpu.VMEM((CH,), jnp.int32),
            pltpu.VMEM((CH,), jnp.int32),
            pltpu.VMEM((CH, D), _F32),
            pltpu.VMEM((CH, D), _F32),
            pltpu.VMEM((CH, D), _F32),
            pltpu.VMEM((CH, D), _F32),
            pltpu.SemaphoreType.DMA,
            pltpu.SemaphoreType.DMA,
        ],
    )


def _full(shape):
    nd = len(shape)
    return pl.BlockSpec(shape, lambda i, _nd=nd: (0,) * _nd)


def _rows(shape):
    nd = len(shape)
    return pl.BlockSpec(shape, lambda i, _nd=nd: (i,) + (0,) * (_nd - 1))


def kernel(x, edge_index, edge_attr, batch, atom_tab, vn_emb, eps, W1, b1,
           bn1_g, bn1_b, W2, b2, bond_tab, bn_g, bn_b, vW1, vb1, vbn1_g,
           vbn1_b, vW2, vb2, vbn2_g, vbn2_b):
    x = x.astype(jnp.int32)
    src = edge_index[0].astype(jnp.int32)
    dst = edge_index[1].astype(jnp.int32)
    batch = batch.astype(jnp.int32)
    atom2 = atom_tab.reshape(9 * 64, D)
    attr3 = edge_attr.astype(jnp.int32).T.reshape(3, E // 128, 128)
    b2d = batch[:, None]
    brow = batch.reshape(NB, 1, B)
    ng = (batch[-1] + 1).reshape(1, 1)
    ii = jnp.arange(512)
    m_sel = jnp.concatenate([
        (ii[:, None] // 64 == jnp.arange(8)[None, :]).astype(_F32),
        ((ii[:, None] // 8) % 8 == jnp.arange(8)[None, :]).astype(_F32),
        (ii[:, None] % 8 == jnp.arange(8)[None, :]).astype(_F32)], axis=1)

    hin = pl.pallas_call(
        _p0_body,
        grid=(NB,),
        in_specs=[_rows((B, 9)), _full((9 * 64, D)), _full((1, D))],
        out_specs=_rows((B, D)),
        out_shape=jax.ShapeDtypeStruct((N, D), _F32),
    )(x, atom2, vn_emb)

    etab_call = pl.pallas_call(
        _etab_body,
        out_shape=jax.ShapeDtypeStruct((512, D), _F32))
    etab = etab_call(bond_tab[0].reshape(24, D), m_sel)
    eidx = pl.pallas_call(
        _eidx_body,
        out_shape=jax.ShapeDtypeStruct((E // 128, 128), jnp.int32),
    )(attr3).reshape(E)

    vn = jnp.broadcast_to(vn_emb[0], (G, D))

    p1_call = pl.pallas_call(
        _p1_body,
        grid=(NB,),
        in_specs=[_rows((B, D)), _rows((B, D)), _rows((B, D)),
                  pl.BlockSpec((1, 1, B), lambda i: (i, 0, 0)),
                  _full((1, 1)),
                  _full((D, 2 * D)), _full((1, 2 * D))],
        out_specs=[_rows((B, 2 * D)), _full((1, 2 * D)), _full((1, 2 * D)),
                   _full((G, D))],
        out_shape=[jax.ShapeDtypeStruct((N, 2 * D), _F32),
                   jax.ShapeDtypeStruct((1, 2 * D), _F32),
                   jax.ShapeDtypeStruct((1, 2 * D), _F32),
                   jax.ShapeDtypeStruct((G, D), _F32)],
    )

    p2_call = pl.pallas_call(
        _p2_body,
        out_shape=[jax.ShapeDtypeStruct((G, D), _F32),
                   jax.ShapeDtypeStruct((512, D), _F32)])

    p3_call = pl.pallas_call(
        _p3_body,
        grid=(NB,),
        in_specs=[_rows((B, 2 * D)), _full((1, 2 * D)), _full((1, 2 * D)),
                  _full((1, 2 * D)), _full((1, 2 * D)),
                  _full((2 * D, D)), _full((1, D))],
        out_specs=[_rows((B, D)), _full((1, D)), _full((1, D))],
        out_shape=[jax.ShapeDtypeStruct((N, D), _F32),
                   jax.ShapeDtypeStruct((1, D), _F32),
                   jax.ShapeDtypeStruct((1, D), _F32)],
    )

    p4_call = pl.pallas_call(
        _p4_body,
        grid=(NB,),
        in_specs=[_rows((B, D)), _full((1, D)), _full((1, D)),
                  _full((1, D)), _full((1, D)), _rows((B, 1)),
                  _full((G, D))],
        out_specs=_rows((B, D)),
        out_shape=jax.ShapeDtypeStruct((N, D), _F32),
    )

    p4f_call = pl.pallas_call(
        _p4f_body,
        grid=(NB,),
        in_specs=[_rows((B, D)), _full((1, D)), _full((1, D)),
                  _full((1, D)), _full((1, D))],
        out_specs=_rows((B, D)),
        out_shape=jax.ShapeDtypeStruct((N, D), _F32),
    )

    out = None
    for l in range(L):
        hin_p = jnp.pad(hin, ((0, NP - N), (0, 0)))
        aggp = _make_edge_call()(hin_p, etab, src, dst, eidx)
        z1, s1, q1, vns = p1_call(
            hin, aggp[0, :N], aggp[1, :N], brow, eps[l].reshape(1, 1),
            W1[l], b1[l][None, :])
        z2, s2, q2 = p3_call(z1, s1, q1, bn1_g[l][None, :],
                             bn1_b[l][None, :], W2[l], b2[l][None, :])
        if l < L - 1:
            vn, etab = p2_call(
                vns, vn, ng, vW1[l], vb1[l][None, :], vbn1_g[l][None, :],
                vbn1_b[l][None, :], vW2[l], vb2[l][None, :],
                vbn2_g[l][None, :], vbn2_b[l][None, :],
                bond_tab[l + 1].reshape(24, D), m_sel)
            hin = p4_call(z2, s2, q2, bn_g[l][None, :], bn_b[l][None, :],
                          b2d, vn)
        else:
            out = p4f_call(z2, s2, q2, bn_g[l][None, :], bn_b[l][None, :])
    return out
